# baseline (speedup 1.0000x reference)
import jax
import jax.numpy as jnp
from jax.experimental import pallas as pl

_BLOCK_S = 512


def _add_kernel(seq_ref, pe_ref, out_ref):
    out_ref[...] = seq_ref[...] + pe_ref[...]


def kernel(sequence, pe):
    batch, seq_len, d_model = sequence.shape
    pe2d = pe[0, :seq_len]

    block_s = _BLOCK_S
    if seq_len % block_s != 0:
        block_s = seq_len
    num_s = seq_len // block_s

    out = pl.pallas_call(
        _add_kernel,
        grid=(num_s, batch),
        in_specs=[
            pl.BlockSpec((1, block_s, d_model), lambda s, b: (b, s, 0)),
            pl.BlockSpec((block_s, d_model), lambda s, b: (s, 0)),
        ],
        out_specs=pl.BlockSpec((1, block_s, d_model), lambda s, b: (b, s, 0)),
        out_shape=jax.ShapeDtypeStruct(sequence.shape, sequence.dtype),
    )(sequence, pe2d)
    return out


# speedup vs baseline: 1.1579x; 1.1579x over previous
import jax
import jax.numpy as jnp
from jax.experimental import pallas as pl

_BLOCK_S = 512


def _add_kernel(seq_ref, pe_ref, out_ref):
    out_ref[...] = seq_ref[...] + pe_ref[...][None, :, :]


def kernel(sequence, pe):
    batch, seq_len, d_model = sequence.shape
    pe2d = pe[0, :seq_len]

    block_s = _BLOCK_S
    if seq_len % block_s != 0:
        block_s = seq_len
    num_s = seq_len // block_s

    out = pl.pallas_call(
        _add_kernel,
        grid=(num_s,),
        in_specs=[
            pl.BlockSpec((batch, block_s, d_model), lambda s: (0, s, 0)),
            pl.BlockSpec((block_s, d_model), lambda s: (s, 0)),
        ],
        out_specs=pl.BlockSpec((batch, block_s, d_model), lambda s: (0, s, 0)),
        out_shape=jax.ShapeDtypeStruct(sequence.shape, sequence.dtype),
    )(sequence, pe2d)
    return out
